# Initial kernel scaffold; baseline (speedup 1.0000x reference)
import jax
import jax.numpy as jnp
from jax.experimental import pallas as pl


def kernel(x):
    raise NotImplementedError("write your pallas kernel here")


# speedup vs baseline: 11.0764x; 11.0764x over previous
import jax
import jax.numpy as jnp
import numpy as np
from jax.experimental import pallas as pl

B, N, C = 8, 2048, 128
CN = 256
KNN = 8
RB = 256
NBLK = N // RB
SQRT_C = np.float32(C ** 0.5)
DEFAULT = jax.lax.Precision.DEFAULT
HIGHEST = jax.lax.Precision.HIGHEST


def _row(v):
    return jnp.swapaxes(v, 0, 1)


def _density_body(xr_ref, xa_ref, noise_ref, dens_ref, dmax_ref):
    i = pl.program_id(1)
    xr = xr_ref[0]
    xa = xa_ref[0]
    sqr = jnp.sum(xr * xr, axis=1, keepdims=True)
    sqa = jnp.sum(xa * xa, axis=1)[None, :]
    g = jax.lax.dot_general(xr, xa, (((1,), (1,)), ((), ())),
                            preferred_element_type=jnp.float32,
                            precision=DEFAULT)
    d2 = sqr + sqa - 2.0 * g
    dist = jnp.sqrt(jnp.maximum(d2, 0.0)) / SQRT_C
    colid = jax.lax.broadcasted_iota(jnp.int32, (RB, N), 1)
    acc = jnp.zeros((RB, 1), jnp.float32)
    dv = dist
    for _ in range(KNN):
        m = jnp.min(dv, axis=1, keepdims=True)
        acc = acc + m * m
        j = jnp.min(jnp.where(dv == m, colid, N), axis=1, keepdims=True)
        dv = jnp.where(colid == j, jnp.inf, dv)
    dens = jnp.exp(-(acc * np.float32(1.0 / KNN)))
    dens_ref[0, :, pl.ds(i * RB, RB)] = (
        _row(dens) + noise_ref[0, :, pl.ds(i * RB, RB)])
    bmax = jnp.max(d2)
    @pl.when(i == 0)
    def _():
        dmax_ref[0, 0, :] = jnp.full((128,), bmax, jnp.float32)
    @pl.when(i != 0)
    def _():
        dmax_ref[0, 0, :] = jnp.maximum(dmax_ref[0, 0, :], bmax)


def _score_body(xr_ref, xa_ref, dens_ref, dmax_ref, score_ref):
    i = pl.program_id(1)
    xr = xr_ref[0]
    xa = xa_ref[0]
    sqr = jnp.sum(xr * xr, axis=1, keepdims=True)
    sqa = jnp.sum(xa * xa, axis=1)[None, :]
    g = jax.lax.dot_general(xr, xa, (((1,), (1,)), ((), ())),
                            preferred_element_type=jnp.float32,
                            precision=DEFAULT)
    d2 = sqr + sqa - 2.0 * g
    densa = dens_ref[0]
    densr = jnp.swapaxes(dens_ref[0, :, pl.ds(i * RB, RB)], 0, 1)
    d2m = jnp.max(dmax_ref[0])
    masked = jnp.where(densa > densr, d2, d2m)
    dmin = jnp.min(masked, axis=1, keepdims=True)
    dist = jnp.sqrt(jnp.maximum(dmin, 0.0)) / SQRT_C
    score_ref[0, :, pl.ds(i * RB, RB)] = _row(dist * densr)


def _topk_body(score_ref, idx_ref):
    s = score_ref[:, 0, :]
    colid = jax.lax.broadcasted_iota(jnp.int32, (B, N), 1)
    slot = jax.lax.broadcasted_iota(jnp.int32, (B, CN), 1)

    def step(t, carry):
        s, idxmat = carry
        m = jnp.max(s, axis=1, keepdims=True)
        j = jnp.min(jnp.where(s == m, colid, N), axis=1, keepdims=True)
        idxmat = jnp.where(slot == t, j, idxmat)
        s = jnp.where(colid == j, -jnp.inf, s)
        return s, idxmat

    _, idxmat = jax.lax.fori_loop(
        0, CN, step, (s, jnp.zeros((B, CN), jnp.int32)))
    idx_ref[:, 0, :] = idxmat


def _merge_body(x_ref, idx_ref, feat_ref):
    x = x_ref[0]
    idxd = idx_ref[0]
    rown = jax.lax.broadcasted_iota(jnp.int32, (N, 1), 0)
    selT = (rown == idxd).astype(jnp.float32)
    dd = (((0,), (0,)), ((), ()))
    xd = jax.lax.dot_general(selT, x, dd,
                             preferred_element_type=jnp.float32,
                             precision=HIGHEST)
    sqa = jnp.sum(x * x, axis=1, keepdims=True)
    sqd = jax.lax.dot_general(sqa, selT, dd,
                              preferred_element_type=jnp.float32,
                              precision=HIGHEST)
    g = jax.lax.dot_general(x, xd, (((1,), (1,)), ((), ())),
                            preferred_element_type=jnp.float32,
                            precision=DEFAULT)
    d2 = sqd + sqa - 2.0 * g
    dsel = jnp.sqrt(jnp.maximum(d2, 0.0)) / SQRT_C
    dsel = jnp.where(rown == idxd, -1.0, dsel)
    m = jnp.min(dsel, axis=1, keepdims=True)
    cid = jax.lax.broadcasted_iota(jnp.int32, (N, CN), 1)
    idxc = jnp.min(jnp.where(dsel == m, cid, CN), axis=1, keepdims=True)
    oh = (idxc == cid).astype(jnp.float32)
    sums = jax.lax.dot_general(oh, x, dd,
                               preferred_element_type=jnp.float32,
                               precision=HIGHEST)
    counts = jax.lax.dot_general(oh, jnp.ones((N, 1), jnp.float32), dd,
                                 preferred_element_type=jnp.float32,
                                 precision=HIGHEST)
    feat_ref[0] = sums / counts


def kernel(x):
    noise = (jax.random.uniform(jax.random.key(1), (B, N), dtype=jnp.float32)
             * 1e-06).reshape(B, 1, N)

    dens, dmax = pl.pallas_call(
        _density_body,
        grid=(B, NBLK),
        in_specs=[
            pl.BlockSpec((1, RB, C), lambda b, i: (b, i, 0)),
            pl.BlockSpec((1, N, C), lambda b, i: (b, 0, 0)),
            pl.BlockSpec((1, 1, N), lambda b, i: (b, 0, 0)),
        ],
        out_specs=[
            pl.BlockSpec((1, 1, N), lambda b, i: (b, 0, 0)),
            pl.BlockSpec((1, 1, 128), lambda b, i: (b, 0, 0)),
        ],
        out_shape=[
            jax.ShapeDtypeStruct((B, 1, N), jnp.float32),
            jax.ShapeDtypeStruct((B, 1, 128), jnp.float32),
        ],
    )(x, x, noise)

    score = pl.pallas_call(
        _score_body,
        grid=(B, NBLK),
        in_specs=[
            pl.BlockSpec((1, RB, C), lambda b, i: (b, i, 0)),
            pl.BlockSpec((1, N, C), lambda b, i: (b, 0, 0)),
            pl.BlockSpec((1, 1, N), lambda b, i: (b, 0, 0)),
            pl.BlockSpec((1, 1, 128), lambda b, i: (b, 0, 0)),
        ],
        out_specs=pl.BlockSpec((1, 1, N), lambda b, i: (b, 0, 0)),
        out_shape=jax.ShapeDtypeStruct((B, 1, N), jnp.float32),
    )(x, x, dens, dmax)

    idxd = pl.pallas_call(
        _topk_body,
        out_shape=jax.ShapeDtypeStruct((B, 1, CN), jnp.int32),
    )(score)

    feat = pl.pallas_call(
        _merge_body,
        grid=(B,),
        in_specs=[
            pl.BlockSpec((1, N, C), lambda b: (b, 0, 0)),
            pl.BlockSpec((1, 1, CN), lambda b: (b, 0, 0)),
        ],
        out_specs=pl.BlockSpec((1, CN, C), lambda b: (b, 0, 0)),
        out_shape=jax.ShapeDtypeStruct((B, CN, C), jnp.float32),
    )(x, idxd)

    return feat


# speedup vs baseline: 14.7814x; 1.3345x over previous
import jax
import jax.numpy as jnp
import numpy as np
from jax.experimental import pallas as pl

B, N, C = 8, 2048, 128
CN = 256
KNN = 8
RB = 256
NBLK = N // RB
SQRT_C = np.float32(C ** 0.5)
DEFAULT = jax.lax.Precision.DEFAULT
HIGHEST = jax.lax.Precision.HIGHEST


def _row(v):
    return jnp.swapaxes(v, 0, 1)


def _density_body(xr_ref, xa_ref, noise_ref, dens_ref, dmax_ref):
    i = pl.program_id(1)
    xr = xr_ref[0]
    xa = xa_ref[0]
    sqr = jnp.sum(xr * xr, axis=1, keepdims=True)
    sqa = jnp.sum(xa * xa, axis=1)[None, :]
    g = jax.lax.dot_general(xr, xa, (((1,), (1,)), ((), ())),
                            preferred_element_type=jnp.float32,
                            precision=DEFAULT)
    d2 = sqr + sqa - 2.0 * g
    W = N // KNN
    e = [d2[:, k * W:(k + 1) * W] for k in range(KNN)]

    def _ce(a, b):
        lo = jnp.minimum(e[a], e[b])
        hi = jnp.maximum(e[a], e[b])
        e[a], e[b] = lo, hi

    for (a, b) in [(0, 1), (2, 3), (4, 5), (6, 7),
                   (0, 2), (1, 3), (4, 6), (5, 7),
                   (1, 2), (5, 6),
                   (0, 4), (1, 5), (2, 6), (3, 7),
                   (2, 4), (3, 5),
                   (1, 2), (3, 4), (5, 6)]:
        _ce(a, b)
    col = jax.lax.broadcasted_iota(jnp.int32, (RB, W), 1)
    acc = jnp.zeros((RB, 1), jnp.float32)
    for _ in range(KNN):
        m = jnp.min(e[0], axis=1, keepdims=True)
        md = jnp.sqrt(jnp.maximum(m, 0.0)) / SQRT_C
        acc = acc + md * md
        p = jnp.min(jnp.where(e[0] == m, col, W), axis=1, keepdims=True)
        cond = col == p
        for k in range(KNN - 1):
            e[k] = jnp.where(cond, e[k + 1], e[k])
        e[KNN - 1] = jnp.where(cond, jnp.inf, e[KNN - 1])
    dens = jnp.exp(-(acc * np.float32(1.0 / KNN)))
    dens_ref[0, :, pl.ds(i * RB, RB)] = (
        _row(dens) + noise_ref[0, :, pl.ds(i * RB, RB)])
    bmax = jnp.max(d2)
    @pl.when(i == 0)
    def _():
        dmax_ref[0, 0, :] = jnp.full((128,), bmax, jnp.float32)
    @pl.when(i != 0)
    def _():
        dmax_ref[0, 0, :] = jnp.maximum(dmax_ref[0, 0, :], bmax)


def _score_body(xr_ref, xa_ref, dens_ref, dmax_ref, score_ref):
    i = pl.program_id(1)
    xr = xr_ref[0]
    xa = xa_ref[0]
    sqr = jnp.sum(xr * xr, axis=1, keepdims=True)
    sqa = jnp.sum(xa * xa, axis=1)[None, :]
    g = jax.lax.dot_general(xr, xa, (((1,), (1,)), ((), ())),
                            preferred_element_type=jnp.float32,
                            precision=DEFAULT)
    d2 = sqr + sqa - 2.0 * g
    densa = dens_ref[0]
    densr = jnp.swapaxes(dens_ref[0, :, pl.ds(i * RB, RB)], 0, 1)
    d2m = jnp.max(dmax_ref[0])
    masked = jnp.where(densa > densr, d2, d2m)
    dmin = jnp.min(masked, axis=1, keepdims=True)
    dist = jnp.sqrt(jnp.maximum(dmin, 0.0)) / SQRT_C
    score_ref[0, :, pl.ds(i * RB, RB)] = _row(dist * densr)


def _topk_body(score_ref, idx_ref):
    s = score_ref[:, 0, :]
    colid = jax.lax.broadcasted_iota(jnp.int32, (B, N), 1)
    slot = jax.lax.broadcasted_iota(jnp.int32, (B, CN), 1)

    def step(t, carry):
        s, idxmat = carry
        m = jnp.max(s, axis=1, keepdims=True)
        j = jnp.min(jnp.where(s == m, colid, N), axis=1, keepdims=True)
        idxmat = jnp.where(slot == t, j, idxmat)
        s = jnp.where(colid == j, -jnp.inf, s)
        return s, idxmat

    _, idxmat = jax.lax.fori_loop(
        0, CN, step, (s, jnp.zeros((B, CN), jnp.int32)))
    idx_ref[:, 0, :] = idxmat


def _merge_body(x_ref, idx_ref, feat_ref):
    x = x_ref[0]
    idxd = idx_ref[0]
    rown = jax.lax.broadcasted_iota(jnp.int32, (N, 1), 0)
    selT = (rown == idxd).astype(jnp.float32)
    dd = (((0,), (0,)), ((), ()))
    xd = jax.lax.dot_general(selT, x, dd,
                             preferred_element_type=jnp.float32,
                             precision=HIGHEST)
    sqa = jnp.sum(x * x, axis=1, keepdims=True)
    sqd = jax.lax.dot_general(sqa, selT, dd,
                              preferred_element_type=jnp.float32,
                              precision=HIGHEST)
    g = jax.lax.dot_general(x, xd, (((1,), (1,)), ((), ())),
                            preferred_element_type=jnp.float32,
                            precision=DEFAULT)
    d2 = sqd + sqa - 2.0 * g
    dsel = jnp.sqrt(jnp.maximum(d2, 0.0)) / SQRT_C
    dsel = jnp.where(rown == idxd, -1.0, dsel)
    m = jnp.min(dsel, axis=1, keepdims=True)
    cid = jax.lax.broadcasted_iota(jnp.int32, (N, CN), 1)
    idxc = jnp.min(jnp.where(dsel == m, cid, CN), axis=1, keepdims=True)
    oh = (idxc == cid).astype(jnp.float32)
    sums = jax.lax.dot_general(oh, x, dd,
                               preferred_element_type=jnp.float32,
                               precision=HIGHEST)
    counts = jax.lax.dot_general(oh, jnp.ones((N, 1), jnp.float32), dd,
                                 preferred_element_type=jnp.float32,
                                 precision=HIGHEST)
    feat_ref[0] = sums / counts


def kernel(x):
    noise = (jax.random.uniform(jax.random.key(1), (B, N), dtype=jnp.float32)
             * 1e-06).reshape(B, 1, N)

    dens, dmax = pl.pallas_call(
        _density_body,
        grid=(B, NBLK),
        in_specs=[
            pl.BlockSpec((1, RB, C), lambda b, i: (b, i, 0)),
            pl.BlockSpec((1, N, C), lambda b, i: (b, 0, 0)),
            pl.BlockSpec((1, 1, N), lambda b, i: (b, 0, 0)),
        ],
        out_specs=[
            pl.BlockSpec((1, 1, N), lambda b, i: (b, 0, 0)),
            pl.BlockSpec((1, 1, 128), lambda b, i: (b, 0, 0)),
        ],
        out_shape=[
            jax.ShapeDtypeStruct((B, 1, N), jnp.float32),
            jax.ShapeDtypeStruct((B, 1, 128), jnp.float32),
        ],
    )(x, x, noise)

    score = pl.pallas_call(
        _score_body,
        grid=(B, NBLK),
        in_specs=[
            pl.BlockSpec((1, RB, C), lambda b, i: (b, i, 0)),
            pl.BlockSpec((1, N, C), lambda b, i: (b, 0, 0)),
            pl.BlockSpec((1, 1, N), lambda b, i: (b, 0, 0)),
            pl.BlockSpec((1, 1, 128), lambda b, i: (b, 0, 0)),
        ],
        out_specs=pl.BlockSpec((1, 1, N), lambda b, i: (b, 0, 0)),
        out_shape=jax.ShapeDtypeStruct((B, 1, N), jnp.float32),
    )(x, x, dens, dmax)

    idxd = pl.pallas_call(
        _topk_body,
        out_shape=jax.ShapeDtypeStruct((B, 1, CN), jnp.int32),
    )(score)

    feat = pl.pallas_call(
        _merge_body,
        grid=(B,),
        in_specs=[
            pl.BlockSpec((1, N, C), lambda b: (b, 0, 0)),
            pl.BlockSpec((1, 1, CN), lambda b: (b, 0, 0)),
        ],
        out_specs=pl.BlockSpec((1, CN, C), lambda b: (b, 0, 0)),
        out_shape=jax.ShapeDtypeStruct((B, CN, C), jnp.float32),
    )(x, idxd)

    return feat


# speedup vs baseline: 15.7831x; 1.0678x over previous
import jax
import jax.numpy as jnp
import numpy as np
from jax.experimental import pallas as pl
from jax.experimental.pallas import tpu as pltpu

B, N, C = 8, 2048, 128
CN = 256
KNN = 8
RB = 256
NBLK = N // RB
SQRT_C = np.float32(C ** 0.5)
DEFAULT = jax.lax.Precision.DEFAULT
HIGHEST = jax.lax.Precision.HIGHEST


def _row(v):
    return jnp.swapaxes(v, 0, 1)


def _knn_density(d2, noise_row):
    W = N // KNN
    e = [d2[:, k * W:(k + 1) * W] for k in range(KNN)]

    def _ce(a, b):
        lo = jnp.minimum(e[a], e[b])
        hi = jnp.maximum(e[a], e[b])
        e[a], e[b] = lo, hi

    for (a, b) in [(0, 1), (2, 3), (4, 5), (6, 7),
                   (0, 2), (1, 3), (4, 6), (5, 7),
                   (1, 2), (5, 6),
                   (0, 4), (1, 5), (2, 6), (3, 7),
                   (2, 4), (3, 5),
                   (1, 2), (3, 4), (5, 6)]:
        _ce(a, b)
    col = jax.lax.broadcasted_iota(jnp.int32, (RB, W), 1)
    acc = jnp.zeros((RB, 1), jnp.float32)
    for _ in range(KNN):
        m = jnp.min(e[0], axis=1, keepdims=True)
        md = jnp.sqrt(jnp.maximum(m, 0.0)) / SQRT_C
        acc = acc + md * md
        p = jnp.min(jnp.where(e[0] == m, col, W), axis=1, keepdims=True)
        cond = col == p
        for k in range(KNN - 1):
            e[k] = jnp.where(cond, e[k + 1], e[k])
        e[KNN - 1] = jnp.where(cond, jnp.inf, e[KNN - 1])
    dens = jnp.exp(-(acc * np.float32(1.0 / KNN)))
    return _row(dens) + noise_row


def _denscore_body(xr_ref, xa_ref, noise_ref, score_ref,
                   d2s_ref, dens_ref, dmax_ref):
    ph = pl.program_id(1)
    i = pl.program_id(2)

    @pl.when(ph == 0)
    def _():
        xr = xr_ref[0]
        xa = xa_ref[0]
        sqr = jnp.sum(xr * xr, axis=1, keepdims=True)
        sqa = jnp.sum(xa * xa, axis=1)[None, :]
        g = jax.lax.dot_general(xr, xa, (((1,), (1,)), ((), ())),
                                preferred_element_type=jnp.float32,
                                precision=DEFAULT)
        d2 = sqr + sqa - 2.0 * g
        d2s_ref[pl.ds(i * RB, RB), :] = d2
        dens_ref[:, pl.ds(i * RB, RB)] = _knn_density(
            d2, noise_ref[0, :, pl.ds(i * RB, RB)])
        bmax = jnp.max(d2)

        @pl.when(i == 0)
        def _():
            dmax_ref[0, :] = jnp.full((128,), bmax, jnp.float32)

        @pl.when(i != 0)
        def _():
            dmax_ref[0, :] = jnp.maximum(dmax_ref[0, :], bmax)

    @pl.when(ph == 1)
    def _():
        d2 = d2s_ref[pl.ds(i * RB, RB), :]
        densa = dens_ref[...]
        densr = jnp.swapaxes(dens_ref[:, pl.ds(i * RB, RB)], 0, 1)
        d2m = jnp.max(dmax_ref[...])
        masked = jnp.where(densa > densr, d2, d2m)
        dmin = jnp.min(masked, axis=1, keepdims=True)
        dist = jnp.sqrt(jnp.maximum(dmin, 0.0)) / SQRT_C
        score_ref[0, :, pl.ds(i * RB, RB)] = _row(dist * densr)


def _topk_body(score_ref, idx_ref):
    s = score_ref[:, 0, :]
    colid = jax.lax.broadcasted_iota(jnp.int32, (B, N), 1)
    slot = jax.lax.broadcasted_iota(jnp.int32, (B, CN), 1)

    def step(t, carry):
        s, idxmat = carry
        m = jnp.max(s, axis=1, keepdims=True)
        j = jnp.min(jnp.where(s == m, colid, N), axis=1, keepdims=True)
        idxmat = jnp.where(slot == t, j, idxmat)
        s = jnp.where(colid == j, -jnp.inf, s)
        return s, idxmat

    _, idxmat = jax.lax.fori_loop(
        0, CN, step, (s, jnp.zeros((B, CN), jnp.int32)))
    idx_ref[:, 0, :] = idxmat


def _merge_body(x_ref, idx_ref, feat_ref):
    x = x_ref[0]
    idxd = idx_ref[0]
    rown = jax.lax.broadcasted_iota(jnp.int32, (N, 1), 0)
    selT = (rown == idxd).astype(jnp.float32)
    dd = (((0,), (0,)), ((), ()))
    xd = jax.lax.dot_general(selT, x, dd,
                             preferred_element_type=jnp.float32,
                             precision=HIGHEST)
    sqa = jnp.sum(x * x, axis=1, keepdims=True)
    sqd = jax.lax.dot_general(sqa, selT, dd,
                              preferred_element_type=jnp.float32,
                              precision=HIGHEST)
    g = jax.lax.dot_general(x, xd, (((1,), (1,)), ((), ())),
                            preferred_element_type=jnp.float32,
                            precision=DEFAULT)
    d2 = sqd + sqa - 2.0 * g
    dsel = jnp.sqrt(jnp.maximum(d2, 0.0)) / SQRT_C
    dsel = jnp.where(rown == idxd, -1.0, dsel)
    m = jnp.min(dsel, axis=1, keepdims=True)
    cid = jax.lax.broadcasted_iota(jnp.int32, (N, CN), 1)
    idxc = jnp.min(jnp.where(dsel == m, cid, CN), axis=1, keepdims=True)
    oh = (idxc == cid).astype(jnp.float32)
    sums = jax.lax.dot_general(oh, x, dd,
                               preferred_element_type=jnp.float32,
                               precision=HIGHEST)
    counts = jax.lax.dot_general(oh, jnp.ones((N, 1), jnp.float32), dd,
                                 preferred_element_type=jnp.float32,
                                 precision=HIGHEST)
    feat_ref[0] = sums / counts


def kernel(x):
    noise = (jax.random.uniform(jax.random.key(1), (B, N), dtype=jnp.float32)
             * 1e-06).reshape(B, 1, N)

    score = pl.pallas_call(
        _denscore_body,
        grid=(B, 2, NBLK),
        in_specs=[
            pl.BlockSpec((1, RB, C), lambda b, ph, i: (b, i * (1 - ph), 0)),
            pl.BlockSpec((1, N, C), lambda b, ph, i: (b, 0, 0)),
            pl.BlockSpec((1, 1, N), lambda b, ph, i: (b, 0, 0)),
        ],
        out_specs=pl.BlockSpec((1, 1, N), lambda b, ph, i: (b, 0, 0)),
        out_shape=jax.ShapeDtypeStruct((B, 1, N), jnp.float32),
        scratch_shapes=[
            pltpu.VMEM((N, N), jnp.float32),
            pltpu.VMEM((1, N), jnp.float32),
            pltpu.VMEM((1, 128), jnp.float32),
        ],
    )(x, x, noise)

    idxd = pl.pallas_call(
        _topk_body,
        out_shape=jax.ShapeDtypeStruct((B, 1, CN), jnp.int32),
    )(score)

    feat = pl.pallas_call(
        _merge_body,
        grid=(B,),
        in_specs=[
            pl.BlockSpec((1, N, C), lambda b: (b, 0, 0)),
            pl.BlockSpec((1, 1, CN), lambda b: (b, 0, 0)),
        ],
        out_specs=pl.BlockSpec((1, CN, C), lambda b: (b, 0, 0)),
        out_shape=jax.ShapeDtypeStruct((B, CN, C), jnp.float32),
    )(x, idxd)

    return feat


# speedup vs baseline: 15.9716x; 1.0119x over previous
import jax
import jax.numpy as jnp
import numpy as np
from jax.experimental import pallas as pl
from jax.experimental.pallas import tpu as pltpu

B, N, C = 8, 2048, 128
CN = 256
KNN = 8
RB = 256
NBLK = N // RB
SQRT_C = np.float32(C ** 0.5)
DEFAULT = jax.lax.Precision.DEFAULT
HIGHEST = jax.lax.Precision.HIGHEST


def _row(v):
    return jnp.swapaxes(v, 0, 1)


def _batcher_pairs(n):
    pairs = []
    p = 1
    while p < n:
        k = p
        while k >= 1:
            for j in range(k % p, n - k, 2 * k):
                for i in range(0, min(k, n - j - k)):
                    if (i + j) // (2 * p) == (i + j + k) // (2 * p):
                        pairs.append((i + j, i + j + k))
            k //= 2
        p *= 2
    return pairs


def _knn_density(d2, noise_row):
    W = N // KNN
    e = [d2[:, k * W:(k + 1) * W] for k in range(KNN)]

    def _ce(a, b):
        lo = jnp.minimum(e[a], e[b])
        hi = jnp.maximum(e[a], e[b])
        e[a], e[b] = lo, hi

    for (a, b) in [(0, 1), (2, 3), (4, 5), (6, 7),
                   (0, 2), (1, 3), (4, 6), (5, 7),
                   (1, 2), (5, 6),
                   (0, 4), (1, 5), (2, 6), (3, 7),
                   (2, 4), (3, 5),
                   (1, 2), (3, 4), (5, 6)]:
        _ce(a, b)
    col = jax.lax.broadcasted_iota(jnp.int32, (RB, W), 1)
    acc = jnp.zeros((RB, 1), jnp.float32)
    for _ in range(KNN):
        m = jnp.min(e[0], axis=1, keepdims=True)
        md = jnp.sqrt(jnp.maximum(m, 0.0)) / SQRT_C
        acc = acc + md * md
        p = jnp.min(jnp.where(e[0] == m, col, W), axis=1, keepdims=True)
        cond = col == p
        for k in range(KNN - 1):
            e[k] = jnp.where(cond, e[k + 1], e[k])
        e[KNN - 1] = jnp.where(cond, jnp.inf, e[KNN - 1])
    dens = jnp.exp(-(acc * np.float32(1.0 / KNN)))
    return _row(dens) + noise_row


def _denscore_body(xr_ref, xa_ref, noise_ref, idx_ref,
                   d2s_ref, dens_ref, dmax_ref, score_ref):
    bb = pl.program_id(0)
    ph = pl.program_id(1)
    i = pl.program_id(2)

    @pl.when(ph == 0)
    def _():
        xr = xr_ref[0]
        xa = xa_ref[0]
        sqr = jnp.sum(xr * xr, axis=1, keepdims=True)
        sqa = jnp.sum(xa * xa, axis=1)[None, :]
        g = jax.lax.dot_general(xr, xa, (((1,), (1,)), ((), ())),
                                preferred_element_type=jnp.float32,
                                precision=DEFAULT)
        d2 = sqr + sqa - 2.0 * g
        d2s_ref[pl.ds(i * RB, RB), :] = d2
        dens_ref[:, pl.ds(i * RB, RB)] = _knn_density(
            d2, noise_ref[0, :, pl.ds(i * RB, RB)])
        bmax = jnp.max(d2)

        @pl.when(i == 0)
        def _():
            dmax_ref[0, :] = jnp.full((128,), bmax, jnp.float32)

        @pl.when(i != 0)
        def _():
            dmax_ref[0, :] = jnp.maximum(dmax_ref[0, :], bmax)

    @pl.when(ph == 1)
    def _():
        d2 = d2s_ref[pl.ds(i * RB, RB), :]
        densa = dens_ref[...]
        densr = jnp.swapaxes(dens_ref[:, pl.ds(i * RB, RB)], 0, 1)
        d2m = jnp.max(dmax_ref[...])
        masked = jnp.where(densa > densr, d2, d2m)
        dmin = jnp.min(masked, axis=1, keepdims=True)
        dist = jnp.sqrt(jnp.maximum(dmin, 0.0)) / SQRT_C
        score_ref[pl.ds(bb, 1), pl.ds(i * RB, RB)] = _row(dist * densr)

        @pl.when((bb == B - 1) & (i == NBLK - 1))
        def _():
            idx_ref[:, 0, :] = _topk_extract(score_ref[...])


def _topk_extract(s):
    SW = 128
    NS = N // SW
    colid = jax.lax.broadcasted_iota(jnp.int32, (B, SW), 1)
    v = [s[:, k * SW:(k + 1) * SW] for k in range(NS)]
    ix = [colid + k * SW for k in range(NS)]
    for (a, b) in _batcher_pairs(NS):
        swap = (v[a] < v[b]) | ((v[a] == v[b]) & (ix[a] > ix[b]))
        va = jnp.where(swap, v[b], v[a])
        vb = jnp.where(swap, v[a], v[b])
        ia = jnp.where(swap, ix[b], ix[a])
        ib = jnp.where(swap, ix[a], ix[b])
        v[a], v[b], ix[a], ix[b] = va, vb, ia, ib
    slot = jax.lax.broadcasted_iota(jnp.int32, (B, CN), 1)

    def step(t, carry):
        v, ix, idxmat = carry
        v, ix = list(v), list(ix)
        m = jnp.max(v[0], axis=1, keepdims=True)
        j = jnp.min(jnp.where(v[0] == m, ix[0], N), axis=1, keepdims=True)
        idxmat = jnp.where(slot == t, j, idxmat)
        cond = ix[0] == j
        for k in range(NS - 1):
            v[k] = jnp.where(cond, v[k + 1], v[k])
            ix[k] = jnp.where(cond, ix[k + 1], ix[k])
        v[NS - 1] = jnp.where(cond, -jnp.inf, v[NS - 1])
        ix[NS - 1] = jnp.where(cond, N, ix[NS - 1])
        return tuple(v), tuple(ix), idxmat

    _, _, idxmat = jax.lax.fori_loop(
        0, CN, step,
        (tuple(v), tuple(ix), jnp.zeros((B, CN), jnp.int32)))
    return idxmat


def _merge_body(x_ref, idx_ref, feat_ref):
    x = x_ref[0]
    idxd = idx_ref[0]
    rown = jax.lax.broadcasted_iota(jnp.int32, (N, 1), 0)
    selT = (rown == idxd).astype(jnp.float32)
    dd = (((0,), (0,)), ((), ()))
    xd = jax.lax.dot_general(selT, x, dd,
                             preferred_element_type=jnp.float32,
                             precision=HIGHEST)
    sqa = jnp.sum(x * x, axis=1, keepdims=True)
    sqd = jax.lax.dot_general(sqa, selT, dd,
                              preferred_element_type=jnp.float32,
                              precision=HIGHEST)
    g = jax.lax.dot_general(x, xd, (((1,), (1,)), ((), ())),
                            preferred_element_type=jnp.float32,
                            precision=DEFAULT)
    d2 = sqd + sqa - 2.0 * g
    dsel = jnp.sqrt(jnp.maximum(d2, 0.0)) / SQRT_C
    dsel = jnp.where(rown == idxd, -1.0, dsel)
    m = jnp.min(dsel, axis=1, keepdims=True)
    cid = jax.lax.broadcasted_iota(jnp.int32, (N, CN), 1)
    idxc = jnp.min(jnp.where(dsel == m, cid, CN), axis=1, keepdims=True)
    oh = (idxc == cid).astype(jnp.float32)
    sums = jax.lax.dot_general(oh, x, dd,
                               preferred_element_type=jnp.float32,
                               precision=HIGHEST)
    counts = jax.lax.dot_general(oh, jnp.ones((N, 1), jnp.float32), dd,
                                 preferred_element_type=jnp.float32,
                                 precision=HIGHEST)
    feat_ref[0] = sums / counts


def kernel(x):
    noise = (jax.random.uniform(jax.random.key(1), (B, N), dtype=jnp.float32)
             * 1e-06).reshape(B, 1, N)

    idxd = pl.pallas_call(
        _denscore_body,
        grid=(B, 2, NBLK),
        in_specs=[
            pl.BlockSpec((1, RB, C), lambda b, ph, i: (b, i * (1 - ph), 0)),
            pl.BlockSpec((1, N, C), lambda b, ph, i: (b, 0, 0)),
            pl.BlockSpec((1, 1, N), lambda b, ph, i: (b, 0, 0)),
        ],
        out_specs=pl.BlockSpec((B, 1, CN), lambda b, ph, i: (0, 0, 0)),
        out_shape=jax.ShapeDtypeStruct((B, 1, CN), jnp.int32),
        scratch_shapes=[
            pltpu.VMEM((N, N), jnp.float32),
            pltpu.VMEM((1, N), jnp.float32),
            pltpu.VMEM((1, 128), jnp.float32),
            pltpu.VMEM((B, N), jnp.float32),
        ],
    )(x, x, noise)

    feat = pl.pallas_call(
        _merge_body,
        grid=(B,),
        in_specs=[
            pl.BlockSpec((1, N, C), lambda b: (b, 0, 0)),
            pl.BlockSpec((1, 1, CN), lambda b: (b, 0, 0)),
        ],
        out_specs=pl.BlockSpec((1, CN, C), lambda b: (b, 0, 0)),
        out_shape=jax.ShapeDtypeStruct((B, CN, C), jnp.float32),
    )(x, idxd)

    return feat


# speedup vs baseline: 16.0056x; 1.0021x over previous
import jax
import jax.numpy as jnp
import numpy as np
from jax.experimental import pallas as pl
from jax.experimental.pallas import tpu as pltpu

B, N, C = 8, 2048, 128
CN = 256
KNN = 8
RB = 256
NBLK = N // RB
SQRT_C = np.float32(C ** 0.5)
DEFAULT = jax.lax.Precision.DEFAULT
HIGHEST = jax.lax.Precision.HIGHEST


def _row(v):
    return jnp.swapaxes(v, 0, 1)


def _batcher_pairs(n):
    pairs = []
    p = 1
    while p < n:
        k = p
        while k >= 1:
            for j in range(k % p, n - k, 2 * k):
                for i in range(0, min(k, n - j - k)):
                    if (i + j) // (2 * p) == (i + j + k) // (2 * p):
                        pairs.append((i + j, i + j + k))
            k //= 2
        p *= 2
    return pairs


def _knn_density(d2, noise_row):
    W = N // KNN
    e = [d2[:, k * W:(k + 1) * W] for k in range(KNN)]

    def _ce(a, b):
        lo = jnp.minimum(e[a], e[b])
        hi = jnp.maximum(e[a], e[b])
        e[a], e[b] = lo, hi

    for (a, b) in [(0, 1), (2, 3), (4, 5), (6, 7),
                   (0, 2), (1, 3), (4, 6), (5, 7),
                   (1, 2), (5, 6),
                   (0, 4), (1, 5), (2, 6), (3, 7),
                   (2, 4), (3, 5),
                   (1, 2), (3, 4), (5, 6)]:
        _ce(a, b)
    col = jax.lax.broadcasted_iota(jnp.int32, (RB, W), 1)
    ms = []
    for t in range(KNN):
        m = jnp.min(e[0], axis=1, keepdims=True)
        ms.append(m)
        if t < KNN - 1:
            p = jnp.min(jnp.where(e[0] == m, col, W), axis=1, keepdims=True)
            cond = col == p
            for k in range(KNN - 1 - t):
                e[k] = jnp.where(cond, e[k + 1], e[k])
    md = jnp.sqrt(jnp.maximum(jnp.concatenate(ms, axis=1), 0.0)) / SQRT_C
    sq8 = md * md
    acc = sq8[:, 0:1]
    for t in range(1, KNN):
        acc = acc + sq8[:, t:t + 1]
    dens = jnp.exp(-(acc * np.float32(1.0 / KNN)))
    return _row(dens) + noise_row


def _denscore_body(xr_ref, xa_ref, noise_ref, idx_ref,
                   d2s_ref, dens_ref, dmax_ref, score_ref):
    bb = pl.program_id(0)
    ph = pl.program_id(1)
    i = pl.program_id(2)

    @pl.when(ph == 0)
    def _():
        xr = xr_ref[0]
        xa = xa_ref[0]
        sqr = jnp.sum(xr * xr, axis=1, keepdims=True)
        sqa = jnp.sum(xa * xa, axis=1)[None, :]
        g = jax.lax.dot_general(xr, xa, (((1,), (1,)), ((), ())),
                                preferred_element_type=jnp.float32,
                                precision=DEFAULT)
        d2 = sqr + sqa - 2.0 * g
        d2s_ref[pl.ds(i * RB, RB), :] = d2
        dens_ref[:, pl.ds(i * RB, RB)] = _knn_density(
            d2, noise_ref[0, :, pl.ds(i * RB, RB)])
        bmax = jnp.max(d2)

        @pl.when(i == 0)
        def _():
            dmax_ref[0, :] = jnp.full((128,), bmax, jnp.float32)

        @pl.when(i != 0)
        def _():
            dmax_ref[0, :] = jnp.maximum(dmax_ref[0, :], bmax)

    @pl.when(ph == 1)
    def _():
        d2 = d2s_ref[pl.ds(i * RB, RB), :]
        densa = dens_ref[...]
        densr = jnp.swapaxes(dens_ref[:, pl.ds(i * RB, RB)], 0, 1)
        d2m = jnp.max(dmax_ref[...])
        masked = jnp.where(densa > densr, d2, d2m)
        dmin = jnp.min(masked, axis=1, keepdims=True)
        dist = jnp.sqrt(jnp.maximum(dmin, 0.0)) / SQRT_C
        score_ref[pl.ds(bb, 1), pl.ds(i * RB, RB)] = _row(dist * densr)

        @pl.when((bb == B - 1) & (i == NBLK - 1))
        def _():
            idx_ref[:, 0, :] = _topk_extract(score_ref[...])


def _topk_extract(s):
    SW = 128
    NS = N // SW
    colid = jax.lax.broadcasted_iota(jnp.int32, (B, SW), 1)
    v = [s[:, k * SW:(k + 1) * SW] for k in range(NS)]
    ix = [colid + k * SW for k in range(NS)]
    for (a, b) in _batcher_pairs(NS):
        swap = (v[a] < v[b]) | ((v[a] == v[b]) & (ix[a] > ix[b]))
        va = jnp.where(swap, v[b], v[a])
        vb = jnp.where(swap, v[a], v[b])
        ia = jnp.where(swap, ix[b], ix[a])
        ib = jnp.where(swap, ix[a], ix[b])
        v[a], v[b], ix[a], ix[b] = va, vb, ia, ib
    slot = jax.lax.broadcasted_iota(jnp.int32, (B, CN), 1)

    def step(t, carry):
        v, ix, idxmat = carry
        v, ix = list(v), list(ix)
        m = jnp.max(v[0], axis=1, keepdims=True)
        j = jnp.min(jnp.where(v[0] == m, ix[0], N), axis=1, keepdims=True)
        idxmat = jnp.where(slot == t, j, idxmat)
        cond = ix[0] == j
        for k in range(NS - 1):
            v[k] = jnp.where(cond, v[k + 1], v[k])
            ix[k] = jnp.where(cond, ix[k + 1], ix[k])
        v[NS - 1] = jnp.where(cond, -jnp.inf, v[NS - 1])
        ix[NS - 1] = jnp.where(cond, N, ix[NS - 1])
        return tuple(v), tuple(ix), idxmat

    _, _, idxmat = jax.lax.fori_loop(
        0, CN, step,
        (tuple(v), tuple(ix), jnp.zeros((B, CN), jnp.int32)))
    return idxmat


def _merge_body(x_ref, idx_ref, feat_ref):
    x = x_ref[0]
    idxd = idx_ref[0]
    rown = jax.lax.broadcasted_iota(jnp.int32, (N, 1), 0)
    selT = (rown == idxd).astype(jnp.float32)
    dd = (((0,), (0,)), ((), ()))
    xd = jax.lax.dot_general(selT, x, dd,
                             preferred_element_type=jnp.float32,
                             precision=HIGHEST)
    sqa = jnp.sum(x * x, axis=1, keepdims=True)
    sqd = jax.lax.dot_general(sqa, selT, dd,
                              preferred_element_type=jnp.float32,
                              precision=HIGHEST)
    g = jax.lax.dot_general(x, xd, (((1,), (1,)), ((), ())),
                            preferred_element_type=jnp.float32,
                            precision=DEFAULT)
    d2 = sqd + sqa - 2.0 * g
    dsel = jnp.sqrt(jnp.maximum(d2, 0.0)) / SQRT_C
    dsel = jnp.where(rown == idxd, -1.0, dsel)
    m = jnp.min(dsel, axis=1, keepdims=True)
    cid = jax.lax.broadcasted_iota(jnp.int32, (N, CN), 1)
    idxc = jnp.min(jnp.where(dsel == m, cid, CN), axis=1, keepdims=True)
    oh = (idxc == cid).astype(jnp.float32)
    sums = jax.lax.dot_general(oh, x, dd,
                               preferred_element_type=jnp.float32,
                               precision=HIGHEST)
    counts = jax.lax.dot_general(oh, jnp.ones((N, 1), jnp.float32), dd,
                                 preferred_element_type=jnp.float32,
                                 precision=HIGHEST)
    feat_ref[0] = sums / counts


def kernel(x):
    noise = (jax.random.uniform(jax.random.key(1), (B, N), dtype=jnp.float32)
             * 1e-06).reshape(B, 1, N)

    idxd = pl.pallas_call(
        _denscore_body,
        grid=(B, 2, NBLK),
        in_specs=[
            pl.BlockSpec((1, RB, C), lambda b, ph, i: (b, i * (1 - ph), 0)),
            pl.BlockSpec((1, N, C), lambda b, ph, i: (b, 0, 0)),
            pl.BlockSpec((1, 1, N), lambda b, ph, i: (b, 0, 0)),
        ],
        out_specs=pl.BlockSpec((B, 1, CN), lambda b, ph, i: (0, 0, 0)),
        out_shape=jax.ShapeDtypeStruct((B, 1, CN), jnp.int32),
        scratch_shapes=[
            pltpu.VMEM((N, N), jnp.float32),
            pltpu.VMEM((1, N), jnp.float32),
            pltpu.VMEM((1, 128), jnp.float32),
            pltpu.VMEM((B, N), jnp.float32),
        ],
    )(x, x, noise)

    feat = pl.pallas_call(
        _merge_body,
        grid=(B,),
        in_specs=[
            pl.BlockSpec((1, N, C), lambda b: (b, 0, 0)),
            pl.BlockSpec((1, 1, CN), lambda b: (b, 0, 0)),
        ],
        out_specs=pl.BlockSpec((1, CN, C), lambda b: (b, 0, 0)),
        out_shape=jax.ShapeDtypeStruct((B, CN, C), jnp.float32),
    )(x, idxd)

    return feat


# speedup vs baseline: 18.8262x; 1.1762x over previous
import jax
import jax.numpy as jnp
import numpy as np
from jax.experimental import pallas as pl
from jax.experimental.pallas import tpu as pltpu

B, N, C = 8, 2048, 128
CN = 256
KNN = 8
RB = 512
NBLK = N // RB
SQRT_C = np.float32(C ** 0.5)
DEFAULT = jax.lax.Precision.DEFAULT
HIGHEST = jax.lax.Precision.HIGHEST


def _row(v):
    return jnp.swapaxes(v, 0, 1)


def _batcher_pairs(n):
    pairs = []
    p = 1
    while p < n:
        k = p
        while k >= 1:
            for j in range(k % p, n - k, 2 * k):
                for i in range(0, min(k, n - j - k)):
                    if (i + j) // (2 * p) == (i + j + k) // (2 * p):
                        pairs.append((i + j, i + j + k))
            k //= 2
        p *= 2
    return pairs


def _knn_density(d2, noise_row):
    W = N // KNN
    e = [d2[:, k * W:(k + 1) * W] for k in range(KNN)]

    def _ce(a, b):
        lo = jnp.minimum(e[a], e[b])
        hi = jnp.maximum(e[a], e[b])
        e[a], e[b] = lo, hi

    for (a, b) in [(0, 1), (2, 3), (4, 5), (6, 7),
                   (0, 2), (1, 3), (4, 6), (5, 7),
                   (1, 2), (5, 6),
                   (0, 4), (1, 5), (2, 6), (3, 7),
                   (2, 4), (3, 5),
                   (1, 2), (3, 4), (5, 6)]:
        _ce(a, b)
    col = jax.lax.broadcasted_iota(jnp.int32, (RB, W), 1)
    ms = []
    for t in range(KNN):
        m = jnp.min(e[0], axis=1, keepdims=True)
        ms.append(m)
        if t < KNN - 1:
            p = jnp.min(jnp.where(e[0] == m, col, W), axis=1, keepdims=True)
            cond = col == p
            for k in range(KNN - 1 - t):
                e[k] = jnp.where(cond, e[k + 1], e[k])
    md = jnp.sqrt(jnp.maximum(jnp.concatenate(ms, axis=1), 0.0)) / SQRT_C
    sq8 = md * md
    acc = sq8[:, 0:1]
    for t in range(1, KNN):
        acc = acc + sq8[:, t:t + 1]
    dens = jnp.exp(-(acc * np.float32(1.0 / KNN)))
    return _row(dens) + noise_row


def _denscore_body(xr_ref, xa_ref, noise_ref, idx_ref,
                   d2s_ref, dens_ref, dmax_ref, score_ref):
    bb = pl.program_id(0)
    ph = pl.program_id(1)
    i = pl.program_id(2)

    @pl.when(ph == 0)
    def _():
        xr = xr_ref[0]
        xa = xa_ref[0]
        sqr = jnp.sum(xr * xr, axis=1, keepdims=True)
        sqa = jnp.sum(xa * xa, axis=1)[None, :]
        g = jax.lax.dot_general(xr, xa, (((1,), (1,)), ((), ())),
                                preferred_element_type=jnp.float32,
                                precision=DEFAULT)
        d2 = sqr + sqa - 2.0 * g
        d2s_ref[pl.ds(i * RB, RB), :] = d2
        dens_ref[:, pl.ds(i * RB, RB)] = _knn_density(
            d2, noise_ref[0, :, pl.ds(i * RB, RB)])
        bmax = jnp.max(d2)

        @pl.when(i == 0)
        def _():
            dmax_ref[0, :] = jnp.full((128,), bmax, jnp.float32)

        @pl.when(i != 0)
        def _():
            dmax_ref[0, :] = jnp.maximum(dmax_ref[0, :], bmax)

    @pl.when(ph == 1)
    def _():
        d2 = d2s_ref[pl.ds(i * RB, RB), :]
        densa = dens_ref[...]
        densr = jnp.swapaxes(dens_ref[:, pl.ds(i * RB, RB)], 0, 1)
        d2m = jnp.max(dmax_ref[...])
        masked = jnp.where(densa > densr, d2, d2m)
        dmin = jnp.min(masked, axis=1, keepdims=True)
        dist = jnp.sqrt(jnp.maximum(dmin, 0.0)) / SQRT_C
        score_ref[pl.ds(bb, 1), pl.ds(i * RB, RB)] = _row(dist * densr)

        @pl.when((bb == B - 1) & (i == NBLK - 1))
        def _():
            idx_ref[:, 0, :] = _topk_extract(score_ref[...])


def _topk_extract(s):
    SW = 128
    NS = N // SW
    colid = jax.lax.broadcasted_iota(jnp.int32, (B, SW), 1)
    v = [s[:, k * SW:(k + 1) * SW] for k in range(NS)]
    ix = [colid + k * SW for k in range(NS)]
    for (a, b) in _batcher_pairs(NS):
        swap = (v[a] < v[b]) | ((v[a] == v[b]) & (ix[a] > ix[b]))
        va = jnp.where(swap, v[b], v[a])
        vb = jnp.where(swap, v[a], v[b])
        ia = jnp.where(swap, ix[b], ix[a])
        ib = jnp.where(swap, ix[a], ix[b])
        v[a], v[b], ix[a], ix[b] = va, vb, ia, ib
    slot = jax.lax.broadcasted_iota(jnp.int32, (B, CN), 1)

    def step(t, carry):
        v, ix, idxmat = carry
        v, ix = list(v), list(ix)
        m = jnp.max(v[0], axis=1, keepdims=True)
        j = jnp.min(jnp.where(v[0] == m, ix[0], N), axis=1, keepdims=True)
        idxmat = jnp.where(slot == t, j, idxmat)
        cond = ix[0] == j
        for k in range(NS - 1):
            v[k] = jnp.where(cond, v[k + 1], v[k])
            ix[k] = jnp.where(cond, ix[k + 1], ix[k])
        v[NS - 1] = jnp.where(cond, -jnp.inf, v[NS - 1])
        ix[NS - 1] = jnp.where(cond, N, ix[NS - 1])
        return tuple(v), tuple(ix), idxmat

    _, _, idxmat = jax.lax.fori_loop(
        0, CN, step,
        (tuple(v), tuple(ix), jnp.zeros((B, CN), jnp.int32)))
    return idxmat


def _merge_body(x_ref, idx_ref, feat_ref):
    x = x_ref[0]
    idxd = idx_ref[0]
    rown = jax.lax.broadcasted_iota(jnp.int32, (N, 1), 0)
    selT = (rown == idxd).astype(jnp.float32)
    dd = (((0,), (0,)), ((), ()))
    xd = jax.lax.dot_general(selT, x, dd,
                             preferred_element_type=jnp.float32,
                             precision=HIGHEST)
    sqa = jnp.sum(x * x, axis=1, keepdims=True)
    sqd = jax.lax.dot_general(sqa, selT, dd,
                              preferred_element_type=jnp.float32,
                              precision=HIGHEST)
    g = jax.lax.dot_general(x, xd, (((1,), (1,)), ((), ())),
                            preferred_element_type=jnp.float32,
                            precision=DEFAULT)
    d2 = sqd + sqa - 2.0 * g
    dsel = jnp.sqrt(jnp.maximum(d2, 0.0)) / SQRT_C
    dsel = jnp.where(rown == idxd, -1.0, dsel)
    m = jnp.min(dsel, axis=1, keepdims=True)
    cid = jax.lax.broadcasted_iota(jnp.int32, (N, CN), 1)
    idxc = jnp.min(jnp.where(dsel == m, cid, CN), axis=1, keepdims=True)
    oh = (idxc == cid).astype(jnp.float32)
    sums = jax.lax.dot_general(oh, x, dd,
                               preferred_element_type=jnp.float32,
                               precision=HIGHEST)
    counts = jax.lax.dot_general(oh, jnp.ones((N, 1), jnp.float32), dd,
                                 preferred_element_type=jnp.float32,
                                 precision=HIGHEST)
    feat_ref[0] = sums / counts


def kernel(x):
    noise = (jax.random.uniform(jax.random.key(1), (B, N), dtype=jnp.float32)
             * 1e-06).reshape(B, 1, N)

    idxd = pl.pallas_call(
        _denscore_body,
        grid=(B, 2, NBLK),
        in_specs=[
            pl.BlockSpec((1, RB, C), lambda b, ph, i: (b, i * (1 - ph), 0)),
            pl.BlockSpec((1, N, C), lambda b, ph, i: (b, 0, 0)),
            pl.BlockSpec((1, 1, N), lambda b, ph, i: (b, 0, 0)),
        ],
        out_specs=pl.BlockSpec((B, 1, CN), lambda b, ph, i: (0, 0, 0)),
        out_shape=jax.ShapeDtypeStruct((B, 1, CN), jnp.int32),
        scratch_shapes=[
            pltpu.VMEM((N, N), jnp.float32),
            pltpu.VMEM((1, N), jnp.float32),
            pltpu.VMEM((1, 128), jnp.float32),
            pltpu.VMEM((B, N), jnp.float32),
        ],
    )(x, x, noise)

    feat = pl.pallas_call(
        _merge_body,
        grid=(B,),
        in_specs=[
            pl.BlockSpec((1, N, C), lambda b: (b, 0, 0)),
            pl.BlockSpec((1, 1, CN), lambda b: (b, 0, 0)),
        ],
        out_specs=pl.BlockSpec((1, CN, C), lambda b: (b, 0, 0)),
        out_shape=jax.ShapeDtypeStruct((B, CN, C), jnp.float32),
    )(x, idxd)

    return feat
